# Initial kernel scaffold; baseline (speedup 1.0000x reference)
import functools

import jax
import jax.numpy as jnp
from jax import lax
from jax.experimental import pallas as pl
from jax.experimental.pallas import tpu as pltpu
from jax.experimental.pallas import tpu_sc as plsc

K = 8192
D = 256
N = 4096
RB = 1024
CB = 1024
INT_MAX = jnp.int32(2147483647)


def _argmin_body(x_ref, c_ref, idx_ref, mn_ref, mi_ref):
    j = pl.program_id(1)
    nj = pl.num_programs(1)

    @pl.when(j == 0)
    def _():
        mn_ref[...] = jnp.full((RB, 1), jnp.inf, jnp.float32)
        mi_ref[...] = jnp.zeros((RB, 1), jnp.int32)

    x = x_ref[...]
    c = c_ref[...]
    xn = jnp.sum(x * x, axis=1, keepdims=True)
    cn = jnp.sum(c * c, axis=1)[None, :]
    mm = lax.dot_general(x, c, (((1,), (1,)), ((), ())),
                         preferred_element_type=jnp.float32)
    d = (xn + cn) - 2.0 * mm
    m_loc = jnp.min(d, axis=1, keepdims=True)
    cols = lax.broadcasted_iota(jnp.int32, (RB, CB), 1) + j * CB
    i_loc = jnp.min(jnp.where(d == m_loc, cols, INT_MAX), axis=1, keepdims=True)
    better = m_loc < mn_ref[...]
    mi_ref[...] = jnp.where(better, i_loc, mi_ref[...])
    mn_ref[...] = jnp.where(better, m_loc, mn_ref[...])

    @pl.when(j == nj - 1)
    def _():
        idx_ref[...] = mi_ref[...][:, 0]


def _argmin_indices(flat, codebook):
    return pl.pallas_call(
        _argmin_body,
        grid=(N // RB, K // CB),
        in_specs=[
            pl.BlockSpec((RB, D), lambda i, j: (i, 0)),
            pl.BlockSpec((CB, D), lambda i, j: (j, 0)),
        ],
        out_specs=pl.BlockSpec((RB,), lambda i, j: (i,)),
        out_shape=jax.ShapeDtypeStruct((N,), jnp.int32),
        scratch_shapes=[
            pltpu.VMEM((RB, 1), jnp.float32),
            pltpu.VMEM((RB, 1), jnp.int32),
        ],
    )(flat, codebook)


def _make_sc_gather():
    info = plsc.get_sparse_core_info()
    nw = info.num_cores * info.num_subcores
    bpw = N // nw
    mesh = plsc.VectorSubcoreMesh(core_axis_name="c", subcore_axis_name="s")

    @functools.partial(
        pl.kernel,
        mesh=mesh,
        out_type=jax.ShapeDtypeStruct((N, D), jnp.float32),
        scratch_types=[
            pltpu.VMEM((bpw,), jnp.int32),
            pltpu.VMEM((bpw, D), jnp.float32),
            pltpu.SemaphoreType.DMA,
        ],
    )
    def gather_k(idx_hbm, table_hbm, out_hbm, idx_v, rows_v, sem):
        wid = lax.axis_index("s") * info.num_cores + lax.axis_index("c")
        base = wid * bpw
        pltpu.sync_copy(idx_hbm.at[pl.ds(base, bpw)], idx_v)
        pltpu.async_copy(table_hbm.at[idx_v], rows_v, sem).wait()
        pltpu.sync_copy(rows_v, out_hbm.at[pl.ds(base, bpw)])

    return gather_k


_sc_gather = _make_sc_gather()


def _finalize_body(x_ref, q_ref, quant_ref, loss_ref):
    x = x_ref[...]
    q = q_ref[...]
    dqx = q - x
    quant_ref[...] = x + dqx
    s = jnp.sum(dqx * dqx)
    loss_ref[0, 0] = 1.25 * (s * (1.0 / (N * D)))


def _finalize(flat, q):
    return pl.pallas_call(
        _finalize_body,
        out_shape=[
            jax.ShapeDtypeStruct((N, D), jnp.float32),
            jax.ShapeDtypeStruct((1, 1), jnp.float32),
        ],
    )(flat, q)


def kernel(inputs, codebook):
    x = jnp.transpose(inputs, (0, 2, 3, 1))
    flat = x.reshape(-1, D)
    idx = _argmin_indices(flat, codebook)
    q = _sc_gather(idx, codebook)
    quant_flat, loss = _finalize(flat, q)
    quant = jnp.transpose(quant_flat.reshape(x.shape), (0, 3, 1, 2))
    return (quant, loss.reshape(()), idx)


# speedup vs baseline: 1.1715x; 1.1715x over previous
import functools

import jax
import jax.numpy as jnp
from jax import lax
from jax.experimental import pallas as pl
from jax.experimental.pallas import tpu as pltpu
from jax.experimental.pallas import tpu_sc as plsc

K = 8192
D = 256
N = 4096
RB = 1024
CB = 1024
INT_MAX = 2147483647


def _argmin_body(x_ref, c_ref, idx_ref, mn_ref, mi_ref):
    j = pl.program_id(1)
    nj = pl.num_programs(1)

    @pl.when(j == 0)
    def _():
        mn_ref[...] = jnp.full((RB, 1), jnp.inf, jnp.float32)
        mi_ref[...] = jnp.zeros((RB, 1), jnp.int32)

    x = x_ref[...]
    c = c_ref[...]
    xn = jnp.sum(x * x, axis=1, keepdims=True)
    cn = jnp.sum(c * c, axis=1)[None, :]
    mm = lax.dot_general(x, c, (((1,), (1,)), ((), ())),
                         preferred_element_type=jnp.float32)
    d = (xn + cn) - 2.0 * mm
    m_loc = jnp.min(d, axis=1, keepdims=True)
    cols = lax.broadcasted_iota(jnp.int32, (RB, CB), 1) + j * CB
    i_loc = jnp.min(jnp.where(d == m_loc, cols, INT_MAX), axis=1, keepdims=True)
    better = m_loc < mn_ref[...]
    mi_ref[...] = jnp.where(better, i_loc, mi_ref[...])
    mn_ref[...] = jnp.where(better, m_loc, mn_ref[...])

    @pl.when(j == nj - 1)
    def _():
        idx_ref[...] = mi_ref[...][:, 0]


def _argmin_indices(flat, codebook):
    return pl.pallas_call(
        _argmin_body,
        grid=(N // RB, K // CB),
        in_specs=[
            pl.BlockSpec((RB, D), lambda i, j: (i, 0)),
            pl.BlockSpec((CB, D), lambda i, j: (j, 0)),
        ],
        out_specs=pl.BlockSpec((RB,), lambda i, j: (i,)),
        out_shape=jax.ShapeDtypeStruct((N,), jnp.int32),
        scratch_shapes=[
            pltpu.VMEM((RB, 1), jnp.float32),
            pltpu.VMEM((RB, 1), jnp.int32),
        ],
    )(flat, codebook)


def _make_sc_gather():
    info = plsc.get_sparse_core_info()
    nw = info.num_cores * info.num_subcores
    bpw = N // nw
    mesh = plsc.VectorSubcoreMesh(core_axis_name="c", subcore_axis_name="s")

    @functools.partial(
        pl.kernel,
        mesh=mesh,
        out_type=jax.ShapeDtypeStruct((N, D), jnp.float32),
        scratch_types=[
            pltpu.VMEM((bpw,), jnp.int32),
            pltpu.VMEM((bpw, D), jnp.float32),
            pltpu.SemaphoreType.DMA,
        ],
    )
    def gather_k(idx_hbm, table_hbm, out_hbm, idx_v, rows_v, sem):
        wid = lax.axis_index("s") * info.num_cores + lax.axis_index("c")
        base = wid * bpw
        pltpu.sync_copy(idx_hbm.at[pl.ds(base, bpw)], idx_v)
        pltpu.async_copy(table_hbm.at[idx_v], rows_v, sem).wait()
        pltpu.sync_copy(rows_v, out_hbm.at[pl.ds(base, bpw)])

    return gather_k


_sc_gather_cache = []


def _sc_gather(idx, table):
    if not _sc_gather_cache:
        _sc_gather_cache.append(_make_sc_gather())
    return _sc_gather_cache[0](idx, table)


def _finalize_body(x_ref, q_ref, quant_ref, loss_ref):
    x = x_ref[...]
    q = q_ref[...]
    dqx = q - x
    quant_ref[...] = x + dqx
    s = jnp.sum(dqx * dqx)
    loss_ref[...] = (1.25 * (s * (1.0 / (N * D)))).reshape(1, 1)


def _finalize(flat, q):
    return pl.pallas_call(
        _finalize_body,
        out_shape=[
            jax.ShapeDtypeStruct((N, D), jnp.float32),
            jax.ShapeDtypeStruct((1, 1), jnp.float32),
        ],
    )(flat, q)


def kernel(inputs, codebook):
    x = jnp.transpose(inputs, (0, 2, 3, 1))
    flat = x.reshape(-1, D)
    idx = _argmin_indices(flat, codebook)
    q = _sc_gather(idx, codebook)
    quant_flat, loss = _finalize(flat, q)
    quant = jnp.transpose(quant_flat.reshape(x.shape), (0, 3, 1, 2))
    return (quant, loss.reshape(()), idx)


# speedup vs baseline: 1.3831x; 1.1806x over previous
import functools

import jax
import jax.numpy as jnp
from jax import lax
from jax.experimental import pallas as pl
from jax.experimental.pallas import tpu as pltpu
from jax.experimental.pallas import tpu_sc as plsc

K = 8192
D = 256
N = 4096
RB = 1024
CB = 4096
INT_MAX = 2147483647


def _argmin_body(x_ref, c_ref, idx_ref, mn_ref, mi_ref):
    j = pl.program_id(1)
    nj = pl.num_programs(1)

    @pl.when(j == 0)
    def _():
        mn_ref[...] = jnp.full((RB, 1), jnp.inf, jnp.float32)
        mi_ref[...] = jnp.zeros((RB, 1), jnp.float32)

    x = x_ref[...]
    c = c_ref[...]
    xn = jnp.sum(x * x, axis=1, keepdims=True)
    cn = jnp.sum(c * c, axis=1)[None, :]
    mm2 = lax.dot_general(x * (-2.0), c, (((1,), (1,)), ((), ())),
                          preferred_element_type=jnp.float32)
    d = (xn + cn) + mm2
    m_loc = jnp.min(d, axis=1, keepdims=True)
    cols = lax.broadcasted_iota(jnp.int32, (1, CB), 1).astype(jnp.float32)
    i_loc = jnp.min(jnp.where(d == m_loc, cols, jnp.inf), axis=1, keepdims=True)
    better = m_loc < mn_ref[...]
    mi_ref[...] = jnp.where(better, i_loc + (j * CB).astype(jnp.float32),
                            mi_ref[...])
    mn_ref[...] = jnp.where(better, m_loc, mn_ref[...])

    @pl.when(j == nj - 1)
    def _():
        idx_ref[...] = mi_ref[...].astype(jnp.int32)


def _argmin_indices(flat, codebook):
    return pl.pallas_call(
        _argmin_body,
        grid=(N // RB, K // CB),
        in_specs=[
            pl.BlockSpec((RB, D), lambda i, j: (i, 0)),
            pl.BlockSpec((CB, D), lambda i, j: (j, 0)),
        ],
        out_specs=pl.BlockSpec((RB, 1), lambda i, j: (i, 0)),
        out_shape=jax.ShapeDtypeStruct((N, 1), jnp.int32),
        scratch_shapes=[
            pltpu.VMEM((RB, 1), jnp.float32),
            pltpu.VMEM((RB, 1), jnp.float32),
        ],
    )(flat, codebook)


def _make_sc_gather():
    info = plsc.get_sparse_core_info()
    nw = info.num_cores * info.num_subcores
    bpw = N // nw
    mesh = plsc.VectorSubcoreMesh(core_axis_name="c", subcore_axis_name="s")

    @functools.partial(
        pl.kernel,
        mesh=mesh,
        out_type=jax.ShapeDtypeStruct((N, D), jnp.float32),
        scratch_types=[
            pltpu.VMEM((bpw,), jnp.int32),
            pltpu.VMEM((bpw, D), jnp.float32),
            pltpu.SemaphoreType.DMA,
        ],
    )
    def gather_k(idx_hbm, table_hbm, out_hbm, idx_v, rows_v, sem):
        wid = lax.axis_index("s") * info.num_cores + lax.axis_index("c")
        base = wid * bpw
        pltpu.sync_copy(idx_hbm.at[pl.ds(base, bpw)], idx_v)
        pltpu.async_copy(table_hbm.at[idx_v], rows_v, sem).wait()
        pltpu.sync_copy(rows_v, out_hbm.at[pl.ds(base, bpw)])

    return gather_k


_sc_gather_cache = []


def _sc_gather(idx, table):
    if not _sc_gather_cache:
        _sc_gather_cache.append(_make_sc_gather())
    return _sc_gather_cache[0](idx, table)


def _finalize_body(x_ref, q_ref, quant_ref, loss_ref):
    x = x_ref[...]
    q = q_ref[...]
    dqx = q - x
    quant_ref[...] = x + dqx
    s = jnp.sum(dqx * dqx)
    loss_ref[...] = (1.25 * (s * (1.0 / (N * D)))).reshape(1, 1)


def _finalize(flat, q):
    return pl.pallas_call(
        _finalize_body,
        out_shape=[
            jax.ShapeDtypeStruct((N, D), jnp.float32),
            jax.ShapeDtypeStruct((1, 1), jnp.float32),
        ],
    )(flat, q)


def kernel(inputs, codebook):
    x = jnp.transpose(inputs, (0, 2, 3, 1))
    flat = x.reshape(-1, D)
    idx = _argmin_indices(flat, codebook).reshape(N)
    q = _sc_gather(idx, codebook)
    quant_flat, loss = _finalize(flat, q)
    quant = jnp.transpose(quant_flat.reshape(x.shape), (0, 3, 1, 2))
    return (quant, loss.reshape(()), idx)
